# Initial kernel scaffold; baseline (speedup 1.0000x reference)
import jax
import jax.numpy as jnp
from jax.experimental import pallas as pl


def kernel(input, table):
    raise NotImplementedError("write your pallas kernel here")


# speedup vs baseline: 1.8740x; 1.8740x over previous
import functools

import jax
import jax.numpy as jnp
from jax import lax
from jax.experimental import pallas as pl
from jax.experimental.pallas import tpu as pltpu
from jax.experimental.pallas import tpu_sc as plsc

_D = 64
_NW = 32
_B = 16384 * 50
_BPW = _B // _NW
_SUB = 128
_SPC = 4
_CH = _SUB * _SPC
_NCH = _BPW // _CH
_NBUF = 2


def _gather_sc(idx_grp, table):
    mesh = plsc.VectorSubcoreMesh(core_axis_name="c", subcore_axis_name="s")

    @functools.partial(
        pl.kernel,
        mesh=mesh,
        out_type=jax.ShapeDtypeStruct((_B, _D), jnp.float32),
        compiler_params=pltpu.CompilerParams(use_tc_tiling_on_sc=False),
        scratch_types=[
            pltpu.VMEM((_NCH * _SPC, _SUB), jnp.int32),
            pltpu.VMEM((_NBUF, _CH, _D), jnp.float32),
            pltpu.SemaphoreType.DMA,
            pltpu.SemaphoreType.DMA,
        ],
    )
    def k(idx_hbm, table_hbm, out_hbm, idx_v, rows_v, gsem0, gsem1):
        gsems = (gsem0, gsem1)
        wid = lax.axis_index("s") * 2 + lax.axis_index("c")
        base = wid * _BPW
        pltpu.sync_copy(idx_hbm.at[wid], idx_v)

        def start_gather(g, b):
            for j in range(_SPC):
                pltpu.async_copy(
                    table_hbm.at[idx_v.at[g * _SPC + j]],
                    rows_v.at[b].at[pl.ds(j * _SUB, _SUB)],
                    gsems[b],
                )

        def wait_gather(g, b):
            for j in range(_SPC):
                pltpu.make_async_copy(
                    table_hbm.at[idx_v.at[g * _SPC + j]],
                    rows_v.at[b].at[pl.ds(j * _SUB, _SUB)],
                    gsems[b],
                ).wait()

        for b in range(_NBUF):
            start_gather(b, b)

        def body(t, carry):
            for b in range(_NBUF):
                g = t * _NBUF + b
                wait_gather(g, b)
                pltpu.sync_copy(
                    rows_v.at[b], out_hbm.at[pl.ds(base + g * _CH, _CH)]
                )

                @pl.when(g + _NBUF < _NCH)
                def _():
                    start_gather(g + _NBUF, b)

            return carry

        lax.fori_loop(0, _NCH // _NBUF, body, 0)

    return k(idx_grp, table)


def kernel(input, table):
    idx = input.reshape(-1).astype(jnp.int32).reshape(_NW, _NCH * _SPC, _SUB)
    out = _gather_sc(idx, table)
    return out.reshape(input.shape[0], input.shape[1], _D)
